# baseline (speedup 1.0000x reference)
import functools

import jax
import jax.numpy as jnp
from jax import lax
from jax.experimental import pallas as pl
from jax.experimental.pallas import tpu as pltpu
from jax.experimental.pallas import tpu_sc as plsc

EMB_DIM = 64
NC = 2
NS = 16
NW = NC * NS
CHUNK = 160
PAIRS = CHUNK // 2
NBUF = 4
LOOKAHEAD = 2


@functools.partial(jax.jit, static_argnames=("total",))
def _embed(table, idx_even, idx_odd, *, total):
    half_per_w = total // 2 // NW
    n_chunks = half_per_w // PAIRS
    n_groups = n_chunks // NBUF

    def body(table_hbm, idxe_hbm, idxo_hbm, out_hbm, idxe_v, idxo_v,
             e0, e1, e2, e3, o0, o1, o2, o3,
             ge0, ge1, ge2, ge3, go0, go1, go2, go3, s0, s1, s2, s3):
        bufe = (e0, e1, e2, e3)
        bufo = (o0, o1, o2, o3)
        gseme = (ge0, ge1, ge2, ge3)
        gsemo = (go0, go1, go2, go3)
        osem = (s0, s1, s2, s3)
        wid = lax.axis_index("s") * NC + lax.axis_index("c")
        base = wid * half_per_w

        pltpu.sync_copy(idxe_hbm.at[pl.ds(base, half_per_w)], idxe_v)
        pltpu.sync_copy(idxo_hbm.at[pl.ds(base, half_per_w)], idxo_v)

        def start_gather(c, j):
            ie = idxe_v.at[pl.ds(c * PAIRS, PAIRS)]
            io = idxo_v.at[pl.ds(c * PAIRS, PAIRS)]
            pltpu.async_copy(table_hbm.at[ie], bufe[j], gseme[j])
            pltpu.async_copy(table_hbm.at[io], bufo[j], gsemo[j])

        def wait_gather(j):
            ie = idxe_v.at[pl.ds(0, PAIRS)]
            pltpu.make_async_copy(table_hbm.at[ie], bufe[j], gseme[j]).wait()
            pltpu.make_async_copy(table_hbm.at[ie], bufo[j], gsemo[j]).wait()

        def _out_copies(c, j):
            p0 = base + c * PAIRS
            srce = bufe[j].at[:, pl.ds(0, EMB_DIM)]
            srco = bufo[j].at[:, pl.ds(0, EMB_DIM)]
            dste = out_hbm.at[pl.ds(p0, PAIRS), 0, :]
            dsto = out_hbm.at[pl.ds(p0, PAIRS), 1, :]
            return (pltpu.make_async_copy(srce, dste, osem[j]),
                    pltpu.make_async_copy(srco, dsto, osem[j]))

        def start_out(c, j):
            a, b = _out_copies(c, j)
            a.start()
            b.start()

        def wait_out(c, j):
            a, b = _out_copies(c, j)
            a.wait()
            b.wait()

        def slot(c, j, first, last):
            jn = (j + LOOKAHEAD) % NBUF
            if not last:
                if not first:
                    wait_out(c + LOOKAHEAD - NBUF, jn)
                start_gather(c + LOOKAHEAD, jn)
            wait_gather(j)
            start_out(c, j)

        for j in range(LOOKAHEAD):
            start_gather(j, j)

        for j in range(NBUF):
            slot(j, j, first=(j < NBUF - LOOKAHEAD), last=False)

        def group(t, _):
            for j in range(NBUF):
                slot(t * NBUF + j, j, first=False, last=False)
            return 0

        lax.fori_loop(1, n_groups - 1, group, 0)

        for j in range(NBUF):
            c = (n_groups - 1) * NBUF + j
            slot(c, j, first=False, last=(c + LOOKAHEAD >= n_chunks))
        for j in range(NBUF):
            wait_out((n_groups - 1) * NBUF + j, j)

    run = pl.kernel(
        body,
        out_type=jax.ShapeDtypeStruct((total // 2, 2, EMB_DIM), jnp.float32),
        mesh=plsc.VectorSubcoreMesh(
            core_axis_name="c", subcore_axis_name="s",
            num_cores=NC, num_subcores=NS,
        ),
        scratch_types=(
            [pltpu.VMEM((half_per_w,), jnp.int32)] * 2
            + [pltpu.VMEM((PAIRS, 2 * EMB_DIM), jnp.float32)] * (2 * NBUF)
            + [pltpu.SemaphoreType.DMA] * (3 * NBUF)
        ),
        compiler_params=pltpu.CompilerParams(use_tc_tiling_on_sc=False),
    )
    return run(table, idx_even, idx_odd)


def kernel(x, emb_weight):
    b, l = x.shape
    total = b * l
    flat = x.reshape(total).astype(jnp.int32)
    tbl = jnp.pad(emb_weight[:-1], ((0, 0), (0, 2 * EMB_DIM - emb_weight.shape[1])))
    out = _embed(tbl, flat[0::2], flat[1::2], total=total)
    return out.reshape(b, l, EMB_DIM)


# speedup vs baseline: 1.0093x; 1.0093x over previous
import functools

import jax
import jax.numpy as jnp
from jax import lax
from jax.experimental import pallas as pl
from jax.experimental.pallas import tpu as pltpu
from jax.experimental.pallas import tpu_sc as plsc

EMB_DIM = 64
NC = 2
NS = 16
NW = NC * NS
CHUNK = 400
NBUF = 4
LOOKAHEAD = 2


@functools.partial(jax.jit, static_argnames=("total",))
def _embed(table, flat_idx, *, total):
    b_per_w = total // NW
    n_chunks = b_per_w // CHUNK
    n_groups = n_chunks // NBUF

    def body(table_hbm, idx_hbm, out_hbm, idx_all,
             r0, r1, r2, r3, g0, g1, g2, g3, o0, o1, o2, o3):
        rows = (r0, r1, r2, r3)
        gsem = (g0, g1, g2, g3)
        osem = (o0, o1, o2, o3)
        wid = lax.axis_index("s") * NC + lax.axis_index("c")
        base = wid * b_per_w

        pltpu.sync_copy(idx_hbm.at[pl.ds(base, b_per_w)], idx_all)

        def start_gather(c, j):
            idx = idx_all.at[pl.ds(c * CHUNK, CHUNK)]
            pltpu.async_copy(table_hbm.at[idx], rows[j], gsem[j])

        def start_out(c, j):
            dst = out_hbm.at[pl.ds(base + c * CHUNK, CHUNK)]
            pltpu.async_copy(rows[j], dst, osem[j])

        def wait_gather(j):
            idx = idx_all.at[pl.ds(0, CHUNK)]
            pltpu.make_async_copy(table_hbm.at[idx], rows[j], gsem[j]).wait()

        def wait_out(c, j):
            dst = out_hbm.at[pl.ds(base + c * CHUNK, CHUNK)]
            pltpu.make_async_copy(rows[j], dst, osem[j]).wait()

        def slot(c, j, first, last):
            jn = (j + LOOKAHEAD) % NBUF
            if not last:
                if not first:
                    wait_out(c + LOOKAHEAD - NBUF, jn)
                start_gather(c + LOOKAHEAD, jn)
            wait_gather(j)
            start_out(c, j)

        for j in range(LOOKAHEAD):
            start_gather(j, j)

        for j in range(NBUF):
            slot(j, j, first=(j < NBUF - LOOKAHEAD), last=False)

        def group(t, _):
            for j in range(NBUF):
                slot(t * NBUF + j, j, first=False, last=False)
            return 0

        lax.fori_loop(1, n_groups - 1, group, 0)

        for j in range(NBUF):
            c = (n_groups - 1) * NBUF + j
            slot(c, j, first=False, last=(c + LOOKAHEAD >= n_chunks))
        for j in range(NBUF):
            wait_out((n_groups - 1) * NBUF + j, j)

    run = pl.kernel(
        body,
        out_type=jax.ShapeDtypeStruct((total, EMB_DIM), jnp.float32),
        mesh=plsc.VectorSubcoreMesh(
            core_axis_name="c", subcore_axis_name="s",
            num_cores=NC, num_subcores=NS,
        ),
        scratch_types=(
            [pltpu.VMEM((b_per_w,), jnp.int32)]
            + [pltpu.VMEM((CHUNK, EMB_DIM), jnp.float32)] * NBUF
            + [pltpu.SemaphoreType.DMA] * (2 * NBUF)
        ),
        compiler_params=pltpu.CompilerParams(use_tc_tiling_on_sc=False),
    )
    return run(table, flat_idx)


def kernel(x, emb_weight):
    b, l = x.shape
    total = b * l
    flat = x.reshape(total).astype(jnp.int32)
    out = _embed(emb_weight, flat, total=total)
    return out.reshape(b, l, EMB_DIM)
